# baseline (speedup 1.0000x reference)
import functools

import jax
import jax.numpy as jnp
from jax.experimental import pallas as pl
from jax.experimental.pallas import tpu as pltpu

_ATOMS = 51
_V_MAX = 10.0
_V_MIN = -10.0
_DELTA = (_V_MAX - _V_MIN) / (_ATOMS - 1)
_BATCH = 16384

_NUM_CHUNKS = 8
_ROWS = _BATCH // _NUM_CHUNKS


def _projection_matrix():
    supports = jnp.linspace(_V_MIN, _V_MAX, _ATOMS).astype(jnp.float32)
    tz = jnp.clip(supports, _V_MIN, _V_MAX)
    b = (tz - _V_MIN) / _DELTA
    l = jnp.floor(b).astype(jnp.int32)
    u = jnp.ceil(b).astype(jnp.int32)
    l = jnp.where((u > 0) & (l == u), l - 1, l)
    u = jnp.where((l < _ATOMS - 1) & (l == u), u + 1, u)
    wl = u.astype(jnp.float32) - b
    wu = b - l.astype(jnp.float32)
    cols = jnp.arange(_ATOMS, dtype=jnp.int32)[None, :]
    p = wl[:, None] * (l[:, None] == cols).astype(jnp.float32)
    p = p + wu[:, None] * (u[:, None] == cols).astype(jnp.float32)
    return p


def _chunk_copy(src_hbm, dst_vmem, i, slot, sem):
    return pltpu.make_async_copy(
        src_hbm.at[pl.ds(i * _ROWS, _ROWS), :],
        dst_vmem.at[slot],
        sem.at[slot],
    )


def _kl_manual(proj_ref, anchor_hbm, feature_hbm, out_ref,
               a_buf, f_buf, a_sem, f_sem):
    proj = proj_ref[...]

    def start(i, slot):
        _chunk_copy(anchor_hbm, a_buf, i, slot, a_sem).start()
        _chunk_copy(feature_hbm, f_buf, i, slot, f_sem).start()

    start(0, 0)

    def body(i, acc):
        slot = jax.lax.rem(i, 2)

        @pl.when(i + 1 < _NUM_CHUNKS)
        def _prefetch():
            start(i + 1, jax.lax.rem(i + 1, 2))

        _chunk_copy(anchor_hbm, a_buf, i, slot, a_sem).wait()
        _chunk_copy(feature_hbm, f_buf, i, slot, f_sem).wait()

        a = a_buf[slot]
        f = f_buf[slot]
        s = jnp.dot(a, proj, preferred_element_type=jnp.float32)
        slog = jnp.where(s == 0.0, 0.0, s * jnp.log(s))
        pointwise = slog - s * jnp.log(f + 1e-16)
        return acc + jnp.sum(pointwise, axis=(0, 1), keepdims=True)

    acc = jax.lax.fori_loop(0, _NUM_CHUNKS, body,
                            jnp.zeros((1, 1), jnp.float32))
    out_ref[...] = acc


@functools.partial(jax.jit, static_argnames=())
def kernel(anchor, feature):
    batch, atoms = anchor.shape
    out = pl.pallas_call(
        _kl_manual,
        in_specs=[
            pl.BlockSpec((atoms, atoms), lambda: (0, 0)),
            pl.BlockSpec(memory_space=pl.ANY),
            pl.BlockSpec(memory_space=pl.ANY),
        ],
        out_specs=pl.BlockSpec((1, 1), lambda: (0, 0)),
        out_shape=jax.ShapeDtypeStruct((1, 1), jnp.float32),
        scratch_shapes=[
            pltpu.VMEM((2, _ROWS, atoms), jnp.float32),
            pltpu.VMEM((2, _ROWS, atoms), jnp.float32),
            pltpu.SemaphoreType.DMA((2,)),
            pltpu.SemaphoreType.DMA((2,)),
        ],
    )(_projection_matrix(), anchor, feature)
    return out[0, 0] / batch
